# baseline (speedup 1.0000x reference)
import functools

import jax
import jax.numpy as jnp
from jax import lax
from jax.experimental import pallas as pl
from jax.experimental.pallas import tpu as pltpu
from jax.experimental.pallas import tpu_sc as plsc

F32 = jnp.float32
INV_SQRT2 = 0.7071067811865476

NC, NS = 2, 16
NW = NC * NS

NE = 160000
NI = 480000
NQ = 960000
KMAX = 16


def _silu(x):
    return x * jax.nn.sigmoid(x)


def _xdown_body(m_ref, rbf_ref, wd_ref, wr_ref, wdn_ref, o_ref):
    xdb = _silu(jnp.dot(m_ref[...], wd_ref[...], preferred_element_type=F32))
    xdb = xdb * jnp.dot(rbf_ref[...], wr_ref[...], preferred_element_type=F32)
    o_ref[...] = _silu(jnp.dot(xdb, wdn_ref[...], preferred_element_type=F32))


def _xdown(m, rbf, wd, wr, wdn):
    B = 2000
    return pl.pallas_call(
        _xdown_body,
        grid=(NE // B,),
        in_specs=[
            pl.BlockSpec((B, 128), lambda i: (i, 0)),
            pl.BlockSpec((B, 16), lambda i: (i, 0)),
            pl.BlockSpec((128, 128), lambda i: (0, 0)),
            pl.BlockSpec((16, 128), lambda i: (0, 0)),
            pl.BlockSpec((128, 32), lambda i: (0, 0)),
        ],
        out_specs=pl.BlockSpec((B, 32), lambda i: (i, 0)),
        out_shape=jax.ShapeDtypeStruct((NE, 32), F32),
    )(m, rbf, wd, wr, wdn)


def _tmul_body(xg_ref, cbf_ref, wc_ref, o_ref):
    o_ref[...] = xg_ref[...] * jnp.dot(
        cbf_ref[...], wc_ref[...], preferred_element_type=F32)


def _tmul(xg, cbf, wc):
    B = 4000
    return pl.pallas_call(
        _tmul_body,
        grid=(NI // B,),
        in_specs=[
            pl.BlockSpec((B, 32), lambda i: (i, 0)),
            pl.BlockSpec((B, 16), lambda i: (i, 0)),
            pl.BlockSpec((16, 32), lambda i: (0, 0)),
        ],
        out_specs=pl.BlockSpec((B, 32), lambda i: (i, 0)),
        out_shape=jax.ShapeDtypeStruct((NI, 32), F32),
    )(xg, cbf, wc)


def _gather_rows(table, idx):
    n, d = idx.shape[0], table.shape[1]
    per_w = n // NW
    nch = (per_w + 127) // 128
    mesh = plsc.VectorSubcoreMesh(core_axis_name="c", subcore_axis_name="s",
                                  num_cores=NC, num_subcores=NS)

    @functools.partial(
        pl.kernel,
        out_type=jax.ShapeDtypeStruct((n, d), F32),
        mesh=mesh,
        compiler_params=pltpu.CompilerParams(use_tc_tiling_on_sc=False),
        scratch_types=[
            pltpu.VMEM((128,), jnp.int32),
            pltpu.VMEM((128, d), F32),
            pltpu.SemaphoreType.DMA,
        ],
    )
    def k(table_hbm, idx_hbm, out_hbm, idx_v, rows_v, sem):
        wid = lax.axis_index("c") * NS + lax.axis_index("s")
        base = wid * per_w

        def chunk(j, c):
            st = base + jnp.minimum(j * 128, per_w - 128)
            pltpu.sync_copy(idx_hbm.at[pl.ds(st, 128)], idx_v)
            pltpu.async_copy(table_hbm.at[idx_v], rows_v, sem).wait()
            pltpu.sync_copy(rows_v, out_hbm.at[pl.ds(st, 128)])
            return c

        lax.fori_loop(0, nch, chunk, 0)

    return k(table, idx)


def _build_m2(t, abd, ca, kidx, bounds, zblk):
    rows_per_w = NE * KMAX // NW
    mesh = plsc.VectorSubcoreMesh(core_axis_name="c", subcore_axis_name="s",
                                  num_cores=NC, num_subcores=NS)
    scratch = [
        pltpu.VMEM((272,), jnp.int32),
        pltpu.VMEM((1024, 32), F32),
        pltpu.VMEM((128,), jnp.int32),
        pltpu.VMEM((128,), jnp.int32),
    ]
    scratch += [pltpu.VMEM((128,), jnp.int32) for _ in range(8)]
    scratch += [pltpu.VMEM((128,), jnp.int32) for _ in range(8)]
    scratch += [pltpu.VMEM((128, 32), F32) for _ in range(8)]
    scratch += [pltpu.SemaphoreType.DMA, pltpu.SemaphoreType.DMA]

    @functools.partial(
        pl.kernel,
        out_type=jax.ShapeDtypeStruct((NE * KMAX, 32), F32),
        mesh=mesh,
        compiler_params=pltpu.CompilerParams(use_tc_tiling_on_sc=False),
        scratch_types=scratch,
    )
    def k(t_hbm, abd_hbm, ca_hbm, kidx_hbm, bounds_hbm, zblk_hbm, m2_hbm,
          bounds_v, zero_v, ca_v, k_v, *rest):
        abd_b = rest[0:8]
        dest_b = rest[8:16]
        row_b = rest[16:24]
        sem, sem2 = rest[24], rest[25]
        wid = lax.axis_index("c") * NS + lax.axis_index("s")

        pltpu.sync_copy(zblk_hbm, zero_v)
        rbase = wid * rows_per_w

        def zc(j, c):
            st = rbase + jnp.minimum(j * 1024, rows_per_w - 1024)
            pltpu.sync_copy(zero_v, m2_hbm.at[pl.ds(st, 1024)])
            return c

        lax.fori_loop(0, rows_per_w // 1024 + 1, zc, 0)
        plsc.subcore_barrier()

        pltpu.sync_copy(bounds_hbm, bounds_v)
        bv = bounds_v[pl.ds(wid * 8, 16)]
        qlo = bv[0]
        qhi = bv[8]
        nch = (qhi - qlo + 7 + 1023) // 1024

        def ch(j, c):
            st0 = qlo + j * 1024
            st = jnp.minimum((st0 // 8) * 8, NQ - 1024)
            for jj in range(8):
                stj = st + jj * 128
                pltpu.sync_copy(ca_hbm.at[pl.ds(stj, 128)], ca_v)
                pltpu.sync_copy(kidx_hbm.at[pl.ds(stj, 128)], k_v)
                pltpu.sync_copy(abd_hbm.at[pl.ds(stj, 128)], abd_b[jj])
                for i in range(8):
                    sl = pl.ds(i * 16, 16)
                    dest_b[jj][sl] = ca_v[sl] * 16 + k_v[sl]
            hs = [pltpu.async_copy(t_hbm.at[abd_b[jj]], row_b[jj], sem)
                  for jj in range(8)]
            for h in hs:
                h.wait()
            hs2 = [pltpu.async_copy(row_b[jj], m2_hbm.at[dest_b[jj]], sem2)
                   for jj in range(8)]
            for h in hs2:
                h.wait()
            return c

        lax.fori_loop(0, nch, ch, 0)

    return k(t, abd, ca, kidx, bounds, zblk)


def _bil_body(m2_ref, sph_ref, w1_ref, w2_ref, x_ref):
    m2 = m2_ref[...]
    sph = sph_ref[...]
    w1 = w1_ref[...]
    b = m2.shape[0]
    acc = jnp.zeros((b, 32, 32), F32)
    for s in range(8):
        sk = jnp.sum(m2 * sph[:, :, s][:, :, None], axis=1)
        acc = acc + sk[:, :, None] * w1[:, :, s][:, None, :]
    x_ref[...] = jnp.dot(acc.reshape(b, 1024), w2_ref[...],
                         preferred_element_type=F32)


def _bilinear(m2, sph, w1, w2f):
    B = 256
    return pl.pallas_call(
        _bil_body,
        grid=(NE // B,),
        in_specs=[
            pl.BlockSpec((B, 16, 32), lambda i: (i, 0, 0)),
            pl.BlockSpec((B, 16, 8), lambda i: (i, 0, 0)),
            pl.BlockSpec((B, 32, 8), lambda i: (i, 0, 0)),
            pl.BlockSpec((1024, 64), lambda i: (0, 0)),
        ],
        out_specs=pl.BlockSpec((B, 64), lambda i: (i, 0)),
        out_shape=jax.ShapeDtypeStruct((NE, 64), F32),
    )(m2, sph, w1, w2f)


def _out_body(x_ref, xs_ref, wca_ref, wac_ref, o_ref):
    xca = _silu(jnp.dot(x_ref[...], wca_ref[...], preferred_element_type=F32))
    xac = _silu(jnp.dot(xs_ref[...], wac_ref[...], preferred_element_type=F32))
    o_ref[...] = (xca + xac) * INV_SQRT2


def _final(x, xs, wca, wac):
    B = 2000
    return pl.pallas_call(
        _out_body,
        grid=(NE // B,),
        in_specs=[
            pl.BlockSpec((B, 64), lambda i: (i, 0)),
            pl.BlockSpec((B, 64), lambda i: (i, 0)),
            pl.BlockSpec((64, 128), lambda i: (0, 0)),
            pl.BlockSpec((64, 128), lambda i: (0, 0)),
        ],
        out_specs=pl.BlockSpec((B, 128), lambda i: (i, 0)),
        out_shape=jax.ShapeDtypeStruct((NE, 128), F32),
    )(x, xs, wca, wac)


def kernel(m, rbf, cbf, sbf_W1, sbf_sph, Kidx4, id_swap, id4_reduce_ca,
           id4_expand_intm_db, id4_expand_abd, W_dense_db, W_mlp_rbf,
           W_mlp_cbf, W_bilinear, W_down, W_up_ca, W_up_ac):
    xd = _xdown(m, rbf, W_dense_db, W_mlp_rbf, W_down)
    xg = _gather_rows(xd, id4_expand_intm_db)
    t = _tmul(xg, cbf, W_mlp_cbf)

    cuts = jnp.arange(NW + 1, dtype=jnp.int32) * (NE // NW)
    bounds = jnp.searchsorted(id4_reduce_ca, cuts, side="left").astype(jnp.int32)
    bounds = jnp.pad(jnp.repeat(bounds, 8), (0, 272 - 8 * (NW + 1)))
    zblk = jnp.zeros((1024, 32), F32)

    m2f = _build_m2(t, id4_expand_abd, id4_reduce_ca, Kidx4, bounds, zblk)
    x = _bilinear(m2f.reshape(NE, KMAX, 32), sbf_sph, sbf_W1,
                  W_bilinear.reshape(1024, 64))
    xs = _gather_rows(x, id_swap)
    return _final(x, xs, W_up_ca, W_up_ac)


# speedup vs baseline: 1.0015x; 1.0015x over previous
import functools

import jax
import jax.numpy as jnp
from jax import lax
from jax.experimental import pallas as pl
from jax.experimental.pallas import tpu as pltpu
from jax.experimental.pallas import tpu_sc as plsc

F32 = jnp.float32
INV_SQRT2 = 0.7071067811865476

NC, NS = 2, 16
NW = NC * NS

NE = 160000
NI = 480000
NQ = 960000
KMAX = 16


def _silu(x):
    return x * jax.nn.sigmoid(x)


def _xdown_body(m_ref, rbf_ref, wd_ref, wr_ref, wdn_ref, o_ref):
    xdb = _silu(jnp.dot(m_ref[...], wd_ref[...], preferred_element_type=F32))
    xdb = xdb * jnp.dot(rbf_ref[...], wr_ref[...], preferred_element_type=F32)
    o_ref[...] = _silu(jnp.dot(xdb, wdn_ref[...], preferred_element_type=F32))


def _xdown(m, rbf, wd, wr, wdn):
    B = 2000
    return pl.pallas_call(
        _xdown_body,
        grid=(NE // B,),
        in_specs=[
            pl.BlockSpec((B, 128), lambda i: (i, 0)),
            pl.BlockSpec((B, 16), lambda i: (i, 0)),
            pl.BlockSpec((128, 128), lambda i: (0, 0)),
            pl.BlockSpec((16, 128), lambda i: (0, 0)),
            pl.BlockSpec((128, 32), lambda i: (0, 0)),
        ],
        out_specs=pl.BlockSpec((B, 32), lambda i: (i, 0)),
        out_shape=jax.ShapeDtypeStruct((NE, 32), F32),
    )(m, rbf, wd, wr, wdn)


def _tmul_body(xg_ref, cbf_ref, wc_ref, o_ref):
    o_ref[...] = xg_ref[...] * jnp.dot(
        cbf_ref[...], wc_ref[...], preferred_element_type=F32)


def _tmul(xg, cbf, wc):
    B = 4000
    return pl.pallas_call(
        _tmul_body,
        grid=(NI // B,),
        in_specs=[
            pl.BlockSpec((B, 32), lambda i: (i, 0)),
            pl.BlockSpec((B, 16), lambda i: (i, 0)),
            pl.BlockSpec((16, 32), lambda i: (0, 0)),
        ],
        out_specs=pl.BlockSpec((B, 32), lambda i: (i, 0)),
        out_shape=jax.ShapeDtypeStruct((NI, 32), F32),
    )(xg, cbf, wc)


def _gather_rows(table, idx):
    n, d = idx.shape[0], table.shape[1]
    per_w = n // NW
    nch = (per_w + 127) // 128
    mesh = plsc.VectorSubcoreMesh(core_axis_name="c", subcore_axis_name="s",
                                  num_cores=NC, num_subcores=NS)

    @functools.partial(
        pl.kernel,
        out_type=jax.ShapeDtypeStruct((n, d), F32),
        mesh=mesh,
        compiler_params=pltpu.CompilerParams(use_tc_tiling_on_sc=False),
        scratch_types=[
            pltpu.VMEM((1024,), jnp.int32),
            pltpu.VMEM((1024, d), F32),
            pltpu.SemaphoreType.DMA,
        ],
    )
    def k(table_hbm, idx_hbm, out_hbm, idx_v, rows_v, sem):
        wid = lax.axis_index("c") * NS + lax.axis_index("s")
        base = wid * per_w

        def chunk(j, c):
            st = base + jnp.minimum(j * 1024, per_w - 1024)
            pltpu.sync_copy(idx_hbm.at[pl.ds(st, 1024)], idx_v)
            hs = [pltpu.async_copy(table_hbm.at[idx_v.at[pl.ds(jj * 128, 128)]],
                                   rows_v.at[pl.ds(jj * 128, 128)], sem)
                  for jj in range(8)]
            for h in hs:
                h.wait()
            pltpu.sync_copy(rows_v, out_hbm.at[pl.ds(st, 1024)])
            return c

        lax.fori_loop(0, (per_w + 1023) // 1024, chunk, 0)

    return k(table, idx)


M2ROWS = NE * KMAX + 4096
TRASH = NE * KMAX


def _build_m2(t, abd, ca, kidx, kidx_nx, bounds, zblk):
    rows_per_w = NE * KMAX // NW
    mesh = plsc.VectorSubcoreMesh(core_axis_name="c", subcore_axis_name="s",
                                  num_cores=NC, num_subcores=NS)
    scratch = [
        pltpu.VMEM((272,), jnp.int32),
        pltpu.VMEM((1024, 32), F32),
        pltpu.VMEM((1024,), jnp.int32),
        pltpu.VMEM((1024,), jnp.int32),
        pltpu.VMEM((1024,), jnp.int32),
        pltpu.VMEM((1024,), jnp.int32),
        pltpu.VMEM((1024, 32), F32),
    ]
    scratch += [pltpu.VMEM((128,), jnp.int32) for _ in range(8)]
    scratch += [pltpu.SemaphoreType.DMA, pltpu.SemaphoreType.DMA]

    @functools.partial(
        pl.kernel,
        out_type=jax.ShapeDtypeStruct((M2ROWS, 32), F32),
        mesh=mesh,
        compiler_params=pltpu.CompilerParams(use_tc_tiling_on_sc=False),
        scratch_types=scratch,
    )
    def k(t_hbm, abd_hbm, ca_hbm, kidx_hbm, kn_hbm, bounds_hbm, zblk_hbm,
          m2_hbm, bounds_v, zero_v, ca_v, k_v, kn_v, abd_v, rows_v, *rest):
        dest_b = rest[0:8]
        sem, sem2 = rest[8], rest[9]
        wid = lax.axis_index("c") * NS + lax.axis_index("s")

        pltpu.sync_copy(zblk_hbm, zero_v)
        rbase = wid * rows_per_w

        def zc(j, c):
            st = rbase + jnp.minimum(j * 1024, rows_per_w - 1024)
            pltpu.sync_copy(zero_v, m2_hbm.at[pl.ds(st, 1024)])
            return c

        lax.fori_loop(0, rows_per_w // 1024 + 1, zc, 0)
        plsc.subcore_barrier()

        pltpu.sync_copy(bounds_hbm, bounds_v)
        bv = bounds_v[pl.ds(wid * 8, 16)]
        qlo = bv[0]
        qhi = bv[8]
        nch = (qhi - qlo + 7 + 1023) // 1024

        def ch(j, c):
            st0 = qlo + j * 1024
            st = jnp.minimum((st0 // 8) * 8, NQ - 1024)
            pltpu.sync_copy(ca_hbm.at[pl.ds(st, 1024)], ca_v)
            pltpu.sync_copy(kidx_hbm.at[pl.ds(st, 1024)], k_v)
            pltpu.sync_copy(kn_hbm.at[pl.ds(st, 1024)], kn_v)
            pltpu.sync_copy(abd_hbm.at[pl.ds(st, 1024)], abd_v)
            for g in range(64):
                sl = pl.ds(g * 16, 16)
                kk = k_v[sl]
                dup = (kk == 15) & (kn_v[sl] == 15)
                dest_b[g // 8][pl.ds((g % 8) * 16, 16)] = jnp.where(
                    dup, TRASH, ca_v[sl] * 16 + kk)
            hs = [pltpu.async_copy(
                t_hbm.at[abd_v.at[pl.ds(jj * 128, 128)]],
                rows_v.at[pl.ds(jj * 128, 128)], sem) for jj in range(8)]
            for h in hs:
                h.wait()
            hs2 = [pltpu.async_copy(rows_v.at[pl.ds(jj * 128, 128)],
                                    m2_hbm.at[dest_b[jj]], sem2)
                   for jj in range(8)]
            for h in hs2:
                h.wait()
            return c

        lax.fori_loop(0, nch, ch, 0)

    return k(t, abd, ca, kidx, kidx_nx, bounds, zblk)


def _bil_body(m2_ref, sph_ref, w1_ref, w2_ref, x_ref):
    m2 = m2_ref[...]
    sph = sph_ref[...]
    w1 = w1_ref[...]
    b = m2.shape[0]
    acc = jnp.zeros((b, 32, 32), F32)
    for s in range(8):
        sk = jnp.sum(m2 * sph[:, :, s][:, :, None], axis=1)
        acc = acc + sk[:, :, None] * w1[:, :, s][:, None, :]
    x_ref[...] = jnp.dot(acc.reshape(b, 1024), w2_ref[...],
                         preferred_element_type=F32)


def _bilinear(m2, sph, w1, w2f):
    B = 256
    return pl.pallas_call(
        _bil_body,
        grid=(NE // B,),
        in_specs=[
            pl.BlockSpec((B, 16, 32), lambda i: (i, 0, 0)),
            pl.BlockSpec((B, 16, 8), lambda i: (i, 0, 0)),
            pl.BlockSpec((B, 32, 8), lambda i: (i, 0, 0)),
            pl.BlockSpec((1024, 64), lambda i: (0, 0)),
        ],
        out_specs=pl.BlockSpec((B, 64), lambda i: (i, 0)),
        out_shape=jax.ShapeDtypeStruct((NE, 64), F32),
    )(m2, sph, w1, w2f)


def _out_body(x_ref, xs_ref, wca_ref, wac_ref, o_ref):
    xca = _silu(jnp.dot(x_ref[...], wca_ref[...], preferred_element_type=F32))
    xac = _silu(jnp.dot(xs_ref[...], wac_ref[...], preferred_element_type=F32))
    o_ref[...] = (xca + xac) * INV_SQRT2


def _final(x, xs, wca, wac):
    B = 2000
    return pl.pallas_call(
        _out_body,
        grid=(NE // B,),
        in_specs=[
            pl.BlockSpec((B, 64), lambda i: (i, 0)),
            pl.BlockSpec((B, 64), lambda i: (i, 0)),
            pl.BlockSpec((64, 128), lambda i: (0, 0)),
            pl.BlockSpec((64, 128), lambda i: (0, 0)),
        ],
        out_specs=pl.BlockSpec((B, 128), lambda i: (i, 0)),
        out_shape=jax.ShapeDtypeStruct((NE, 128), F32),
    )(x, xs, wca, wac)


def kernel(m, rbf, cbf, sbf_W1, sbf_sph, Kidx4, id_swap, id4_reduce_ca,
           id4_expand_intm_db, id4_expand_abd, W_dense_db, W_mlp_rbf,
           W_mlp_cbf, W_bilinear, W_down, W_up_ca, W_up_ac):
    xd = _xdown(m, rbf, W_dense_db, W_mlp_rbf, W_down)
    xg = _gather_rows(xd, id4_expand_intm_db)
    t = _tmul(xg, cbf, W_mlp_cbf)

    cuts = jnp.arange(NW + 1, dtype=jnp.int32) * (NE // NW)
    bounds = jnp.searchsorted(id4_reduce_ca, cuts, side="left").astype(jnp.int32)
    bounds = jnp.pad(jnp.repeat(bounds, 8), (0, 272 - 8 * (NW + 1)))
    zblk = jnp.zeros((1024, 32), F32)

    kidx_nx = jnp.concatenate([Kidx4[1:], jnp.zeros((1,), jnp.int32)])
    m2f = _build_m2(t, id4_expand_abd, id4_reduce_ca, Kidx4, kidx_nx,
                    bounds, zblk)
    x = _bilinear(m2f.reshape(M2ROWS // KMAX, KMAX, 32), sbf_sph, sbf_W1,
                  W_bilinear.reshape(1024, 64))
    xs = _gather_rows(x, id_swap)
    return _final(x, xs, W_up_ca, W_up_ac)


# speedup vs baseline: 6.9803x; 6.9698x over previous
import functools

import jax
import jax.numpy as jnp
from jax import lax
from jax.experimental import pallas as pl
from jax.experimental.pallas import tpu as pltpu
from jax.experimental.pallas import tpu_sc as plsc

F32 = jnp.float32
INV_SQRT2 = 0.7071067811865476

NC, NS = 2, 16
NW = NC * NS

NE = 160000
NI = 480000
NQ = 960000
KMAX = 16


def _silu(x):
    return x * jax.nn.sigmoid(x)


def _xdown_body(m_ref, rbf_ref, wd_ref, wr_ref, wdn_ref, o_ref):
    xdb = _silu(jnp.dot(m_ref[...], wd_ref[...], preferred_element_type=F32))
    xdb = xdb * jnp.dot(rbf_ref[...], wr_ref[...], preferred_element_type=F32)
    o_ref[...] = _silu(jnp.dot(xdb, wdn_ref[...], preferred_element_type=F32))


def _xdown(m, rbf, wd, wr, wdn):
    B = 2000
    return pl.pallas_call(
        _xdown_body,
        grid=(NE // B,),
        in_specs=[
            pl.BlockSpec((B, 128), lambda i: (i, 0)),
            pl.BlockSpec((B, 16), lambda i: (i, 0)),
            pl.BlockSpec((128, 128), lambda i: (0, 0)),
            pl.BlockSpec((16, 128), lambda i: (0, 0)),
            pl.BlockSpec((128, 32), lambda i: (0, 0)),
        ],
        out_specs=pl.BlockSpec((B, 32), lambda i: (i, 0)),
        out_shape=jax.ShapeDtypeStruct((NE, 32), F32),
    )(m, rbf, wd, wr, wdn)


def _tmul_body(xg_ref, cbf_ref, wc_ref, o_ref):
    o_ref[...] = xg_ref[...] * jnp.dot(
        cbf_ref[...], wc_ref[...], preferred_element_type=F32)


def _tmul(xg, cbf, wc):
    B = 4000
    return pl.pallas_call(
        _tmul_body,
        grid=(NI // B,),
        in_specs=[
            pl.BlockSpec((B, 32), lambda i: (i, 0)),
            pl.BlockSpec((B, 16), lambda i: (i, 0)),
            pl.BlockSpec((16, 32), lambda i: (0, 0)),
        ],
        out_specs=pl.BlockSpec((B, 32), lambda i: (i, 0)),
        out_shape=jax.ShapeDtypeStruct((NI, 32), F32),
    )(xg, cbf, wc)


def _gather_rows(table, idx):
    n, d = idx.shape[0], table.shape[1]
    per_w = n // NW
    nch = (per_w + 127) // 128
    mesh = plsc.VectorSubcoreMesh(core_axis_name="c", subcore_axis_name="s",
                                  num_cores=NC, num_subcores=NS)

    @functools.partial(
        pl.kernel,
        out_type=jax.ShapeDtypeStruct((n, d), F32),
        mesh=mesh,
        compiler_params=pltpu.CompilerParams(use_tc_tiling_on_sc=False),
        scratch_types=[
            pltpu.VMEM((1024,), jnp.int32),
            pltpu.VMEM((1024, d), F32),
            pltpu.SemaphoreType.DMA,
        ],
    )
    def k(table_hbm, idx_hbm, out_hbm, idx_v, rows_v, sem):
        wid = lax.axis_index("c") * NS + lax.axis_index("s")
        base = wid * per_w

        def chunk(j, c):
            st = base + jnp.minimum(j * 1024, per_w - 1024)
            pltpu.sync_copy(idx_hbm.at[pl.ds(st, 1024)], idx_v)
            hs = [pltpu.async_copy(table_hbm.at[idx_v.at[pl.ds(jj * 128, 128)]],
                                   rows_v.at[pl.ds(jj * 128, 128)], sem)
                  for jj in range(8)]
            for h in hs:
                h.wait()
            pltpu.sync_copy(rows_v, out_hbm.at[pl.ds(st, 1024)])
            return c

        lax.fori_loop(0, (per_w + 1023) // 1024, chunk, 0)

    return k(table, idx)


M2ROWS = NE * KMAX + 4096
TRASH = NE * KMAX


def _build_m2(t, abd, ca, kidx, kidx_nx, bounds, zblk):
    rows_per_w = NE * KMAX // NW
    mesh = plsc.VectorSubcoreMesh(core_axis_name="c", subcore_axis_name="s",
                                  num_cores=NC, num_subcores=NS)
    scratch = [
        pltpu.VMEM((272,), jnp.int32),
        pltpu.VMEM((1024, 32), F32),
        pltpu.VMEM((1024,), jnp.int32),
        pltpu.VMEM((1024,), jnp.int32),
        pltpu.VMEM((1024,), jnp.int32),
        pltpu.VMEM((1024,), jnp.int32),
        pltpu.VMEM((1024, 32), F32),
    ]
    scratch += [pltpu.VMEM((128,), jnp.int32) for _ in range(8)]
    scratch += [pltpu.SemaphoreType.DMA, pltpu.SemaphoreType.DMA]

    @functools.partial(
        pl.kernel,
        out_type=jax.ShapeDtypeStruct((M2ROWS, 32), F32),
        mesh=mesh,
        compiler_params=pltpu.CompilerParams(use_tc_tiling_on_sc=False),
        scratch_types=scratch,
    )
    def k(t_hbm, abd_hbm, ca_hbm, kidx_hbm, kn_hbm, bounds_hbm, zblk_hbm,
          m2_hbm, bounds_v, zero_v, ca_v, k_v, kn_v, abd_v, rows_v, *rest):
        dest_b = rest[0:8]
        sem, sem2 = rest[8], rest[9]
        wid = lax.axis_index("c") * NS + lax.axis_index("s")

        pltpu.sync_copy(zblk_hbm, zero_v)
        rbase = wid * rows_per_w

        def zc(j, c):
            st = rbase + jnp.minimum(j * 1024, rows_per_w - 1024)
            pltpu.sync_copy(zero_v, m2_hbm.at[pl.ds(st, 1024)])
            return c

        lax.fori_loop(0, rows_per_w // 1024 + 1, zc, 0)
        plsc.subcore_barrier()

        pltpu.sync_copy(bounds_hbm, bounds_v)
        bv = bounds_v[pl.ds(wid * 8, 16)]
        qlo = bv[0]
        qhi = bv[8]
        nch = (qhi - qlo + 7 + 1023) // 1024

        def ch(j, c):
            st0 = qlo + j * 1024
            st = jnp.minimum((st0 // 8) * 8, NQ - 1024)
            pltpu.sync_copy(ca_hbm.at[pl.ds(st, 1024)], ca_v)
            pltpu.sync_copy(kidx_hbm.at[pl.ds(st, 1024)], k_v)
            pltpu.sync_copy(kn_hbm.at[pl.ds(st, 1024)], kn_v)
            pltpu.sync_copy(abd_hbm.at[pl.ds(st, 1024)], abd_v)
            for g in range(64):
                sl = pl.ds(g * 16, 16)
                kk = k_v[sl]
                dup = (kk == 15) & (kn_v[sl] == 15)
                dest_b[g // 8][pl.ds((g % 8) * 16, 16)] = jnp.where(
                    dup, TRASH, ca_v[sl] * 16 + kk)
            hs = [pltpu.async_copy(
                t_hbm.at[abd_v.at[pl.ds(jj * 128, 128)]],
                rows_v.at[pl.ds(jj * 128, 128)], sem) for jj in range(8)]
            for h in hs:
                h.wait()
            hs2 = [pltpu.async_copy(rows_v.at[pl.ds(jj * 128, 128)],
                                    m2_hbm.at[dest_b[jj]], sem2)
                   for jj in range(8)]
            for h in hs2:
                h.wait()
            return c

        lax.fori_loop(0, nch, ch, 0)

    return k(t, abd, ca, kidx, kidx_nx, bounds, zblk)


def _bil_body(m2_ref, spht_ref, w1t_ref, w2t_ref, x_ref):
    b = m2_ref.shape[0]
    m2t = m2_ref[...].T
    spht = spht_ref[...]
    w1t = w1t_ref[...]
    skt = []
    for s in range(8):
        a = jnp.zeros((32, b), F32)
        for kk in range(16):
            a = a + m2t[kk * 32:(kk + 1) * 32, :] * spht[kk * 8 + s, :][None, :]
        skt.append(a)
    rws = []
    for i in range(32):
        r = jnp.zeros((32, b), F32)
        for s in range(8):
            r = r + skt[s] * w1t[i * 8 + s, :][None, :]
        rws.append(r)
    rwt = jnp.concatenate(rws, axis=0)
    xt = jnp.dot(w2t_ref[...], rwt, preferred_element_type=F32)
    x_ref[...] = xt.T


def _bilinear(m2r, spht, w1t, w2t):
    B = 256
    return pl.pallas_call(
        _bil_body,
        grid=(NE // B,),
        in_specs=[
            pl.BlockSpec((B, 512), lambda i: (i, 0)),
            pl.BlockSpec((128, B), lambda i: (0, i)),
            pl.BlockSpec((256, B), lambda i: (0, i)),
            pl.BlockSpec((64, 1024), lambda i: (0, 0)),
        ],
        out_specs=pl.BlockSpec((B, 64), lambda i: (i, 0)),
        out_shape=jax.ShapeDtypeStruct((NE, 64), F32),
    )(m2r, spht, w1t, w2t)


def _out_body(x_ref, xs_ref, wca_ref, wac_ref, o_ref):
    xca = _silu(jnp.dot(x_ref[...], wca_ref[...], preferred_element_type=F32))
    xac = _silu(jnp.dot(xs_ref[...], wac_ref[...], preferred_element_type=F32))
    o_ref[...] = (xca + xac) * INV_SQRT2


def _final(x, xs, wca, wac):
    B = 2000
    return pl.pallas_call(
        _out_body,
        grid=(NE // B,),
        in_specs=[
            pl.BlockSpec((B, 64), lambda i: (i, 0)),
            pl.BlockSpec((B, 64), lambda i: (i, 0)),
            pl.BlockSpec((64, 128), lambda i: (0, 0)),
            pl.BlockSpec((64, 128), lambda i: (0, 0)),
        ],
        out_specs=pl.BlockSpec((B, 128), lambda i: (i, 0)),
        out_shape=jax.ShapeDtypeStruct((NE, 128), F32),
    )(x, xs, wca, wac)


def kernel(m, rbf, cbf, sbf_W1, sbf_sph, Kidx4, id_swap, id4_reduce_ca,
           id4_expand_intm_db, id4_expand_abd, W_dense_db, W_mlp_rbf,
           W_mlp_cbf, W_bilinear, W_down, W_up_ca, W_up_ac):
    xd = _xdown(m, rbf, W_dense_db, W_mlp_rbf, W_down)
    xg = _gather_rows(xd, id4_expand_intm_db)
    t = _tmul(xg, cbf, W_mlp_cbf)

    cuts = jnp.arange(NW + 1, dtype=jnp.int32) * (NE // NW)
    bounds = jnp.searchsorted(id4_reduce_ca, cuts, side="left").astype(jnp.int32)
    bounds = jnp.pad(jnp.repeat(bounds, 8), (0, 272 - 8 * (NW + 1)))
    zblk = jnp.zeros((1024, 32), F32)

    kidx_nx = jnp.concatenate([Kidx4[1:], jnp.zeros((1,), jnp.int32)])
    m2f = _build_m2(t, id4_expand_abd, id4_reduce_ca, Kidx4, kidx_nx,
                    bounds, zblk)
    spht = sbf_sph.reshape(NE, 128).T
    w1t = sbf_W1.reshape(NE, 256).T
    w2t = W_bilinear.transpose(1, 0, 2).reshape(1024, 64).T
    x = _bilinear(m2f.reshape(M2ROWS // KMAX, 512), spht, w1t, w2t)
    xs = _gather_rows(x, id_swap)
    return _final(x, xs, W_up_ca, W_up_ac)
